# Initial kernel scaffold; baseline (speedup 1.0000x reference)
import jax
import jax.numpy as jnp
from jax.experimental import pallas as pl


def kernel(x, edge_index, W1, b1, W2, b2, W3, b3, W4, b4):
    raise NotImplementedError("write your pallas kernel here")


# speedup vs baseline: 15.2190x; 15.2190x over previous
import functools

import jax
import jax.numpy as jnp
from jax import lax
from jax.experimental import pallas as pl
from jax.experimental.pallas import tpu as pltpu
from jax.experimental.pallas import tpu_sc as plsc

_N = 10000
_D = 128
_H = 64
_E = 320000

_NC = 2
_NS = 16
_NW = _NC * _NS
_CW = 128
_NCH = 80
_EPW = _NCH * _CW
_EPAD = _NW * _EPW
_NDUMP = 8
_NIT = 10
_RPT = _N // _NIT


def _sc_aggregate(y, zeros_n, src3, dst3):
    mesh = plsc.VectorSubcoreMesh(
        core_axis_name="c", subcore_axis_name="s", num_cores=_NC, num_subcores=_NS
    )
    nacc = _N + _NDUMP

    @functools.partial(
        pl.kernel,
        mesh=mesh,
        out_type=jax.ShapeDtypeStruct((_NC, _N, _H), jnp.float32),
        scratch_types=[
            pltpu.VMEM((_NCH, _CW), jnp.int32),
            pltpu.VMEM((_NCH, _CW), jnp.int32),
            pltpu.VMEM((2, _CW, _H), jnp.float32),
            pltpu.VMEM_SHARED((nacc, _H), jnp.float32),
            pltpu.SemaphoreType.DMA,
            pltpu.SemaphoreType.DMA,
        ],
        compiler_params=pltpu.CompilerParams(use_tc_tiling_on_sc=False),
    )
    def agg(y_hbm, zero_hbm, src_hbm, dst_hbm, out_hbm, src_v, dst_v, rows_v, acc, sem0, sem1):
        c = lax.axis_index("c")
        s = lax.axis_index("s")
        w = s * _NC + c
        r0 = s * _RPT

        pltpu.sync_copy(src_hbm.at[w], src_v)
        pltpu.sync_copy(dst_hbm.at[w], dst_v)

        @pl.when(jnp.logical_and(c == 0, s < _NIT))
        def _():
            pltpu.sync_copy(y_hbm.at[pl.ds(r0, _RPT)], acc.at[pl.ds(r0, _RPT)])

        @pl.when(jnp.logical_and(c == 1, s < _NIT))
        def _():
            pltpu.sync_copy(zero_hbm.at[pl.ds(r0, _RPT)], acc.at[pl.ds(r0, _RPT)])

        plsc.subcore_barrier()

        pltpu.async_copy(y_hbm.at[src_v.at[0]], rows_v.at[0], sem0)

        def body(g, carry):
            j0 = 2 * g
            j1 = j0 + 1
            pltpu.async_copy(y_hbm.at[src_v.at[j1]], rows_v.at[1], sem1)
            pltpu.make_async_copy(y_hbm.at[src_v.at[j0]], rows_v.at[0], sem0).wait()
            pltpu.sync_copy(rows_v.at[0], acc.at[dst_v.at[j0]], add=True)

            @pl.when(j0 + 2 < _NCH)
            def _():
                pltpu.async_copy(y_hbm.at[src_v.at[j0 + 2]], rows_v.at[0], sem0)

            pltpu.make_async_copy(y_hbm.at[src_v.at[j1]], rows_v.at[1], sem1).wait()
            pltpu.sync_copy(rows_v.at[1], acc.at[dst_v.at[j1]], add=True)
            return carry

        lax.fori_loop(0, _NCH // 2, body, 0)

        plsc.subcore_barrier()

        @pl.when(s < _NIT)
        def _():
            pltpu.sync_copy(acc.at[pl.ds(r0, _RPT)], out_hbm.at[c, pl.ds(r0, _RPT)])

    return agg(y, zeros_n, src3, dst3)


def _mm_a(x, w1):
    def body(x_ref, w_ref, o_ref):
        o_ref[...] = jnp.dot(x_ref[...], w_ref[...], preferred_element_type=jnp.float32)

    return pl.pallas_call(
        body,
        grid=(5,),
        in_specs=[
            pl.BlockSpec((_N // 5, _D), lambda i: (i, 0)),
            pl.BlockSpec((_D, _H), lambda i: (0, 0)),
        ],
        out_specs=pl.BlockSpec((_N // 5, _H), lambda i: (i, 0)),
        out_shape=jax.ShapeDtypeStruct((_N, _H), jnp.float32),
    )(x, w1)


def _mlp_b(p, b1, w2, b2, w3):
    def body(p0_ref, p1_ref, b1_ref, w2_ref, b2_ref, w3_ref, o_ref):
        h = jnp.maximum(p0_ref[0] + p1_ref[0] + b1_ref[...], 0.0)
        h = jnp.maximum(
            jnp.dot(h, w2_ref[...], preferred_element_type=jnp.float32) + b2_ref[...], 0.0
        )
        o_ref[...] = jnp.dot(h, w3_ref[...], preferred_element_type=jnp.float32)

    bn = _N // 5
    return pl.pallas_call(
        body,
        grid=(5,),
        in_specs=[
            pl.BlockSpec((1, bn, _H), lambda i: (0, i, 0)),
            pl.BlockSpec((1, bn, _H), lambda i: (1, i, 0)),
            pl.BlockSpec((1, _H), lambda i: (0, 0)),
            pl.BlockSpec((_H, _H), lambda i: (0, 0)),
            pl.BlockSpec((1, _H), lambda i: (0, 0)),
            pl.BlockSpec((_H, _H), lambda i: (0, 0)),
        ],
        out_specs=pl.BlockSpec((bn, _H), lambda i: (i, 0)),
        out_shape=jax.ShapeDtypeStruct((_N, _H), jnp.float32),
    )(p, p, b1.reshape(1, _H), w2, b2.reshape(1, _H), w3)


def _mlp_c(q, b3, w4, b4):
    def body(q0_ref, q1_ref, b3_ref, w4_ref, b4_ref, o_ref):
        h = jnp.maximum(q0_ref[0] + q1_ref[0] + b3_ref[...], 0.0)
        o_ref[...] = (
            jnp.dot(h, w4_ref[...], preferred_element_type=jnp.float32) + b4_ref[...]
        )

    bn = _N // 5
    return pl.pallas_call(
        body,
        grid=(5,),
        in_specs=[
            pl.BlockSpec((1, bn, _H), lambda i: (0, i, 0)),
            pl.BlockSpec((1, bn, _H), lambda i: (1, i, 0)),
            pl.BlockSpec((1, _H), lambda i: (0, 0)),
            pl.BlockSpec((_H, _D), lambda i: (0, 0)),
            pl.BlockSpec((1, _D), lambda i: (0, 0)),
        ],
        out_specs=pl.BlockSpec((bn, _D), lambda i: (i, 0)),
        out_shape=jax.ShapeDtypeStruct((_N, _D), jnp.float32),
    )(q, q, b3.reshape(1, _H), w4, b4.reshape(1, _D))


def kernel(x, edge_index, W1, b1, W2, b2, W3, b3, W4, b4):
    npad = _EPAD - _E
    pad_iota = jnp.arange(npad, dtype=jnp.int32)
    pad_src = pad_iota % _N
    pad_dst = _N + (pad_iota % _NDUMP)
    src3 = jnp.concatenate([edge_index[0], pad_src]).reshape(_NW, _NCH, _CW)
    dst3 = jnp.concatenate([edge_index[1], pad_dst]).reshape(_NW, _NCH, _CW)
    zeros_n = jnp.zeros((_N, _H), jnp.float32)

    y = _mm_a(x, W1)
    p = _sc_aggregate(y, zeros_n, src3, dst3)
    z = _mlp_b(p, b1, W2, b2, W3)
    q = _sc_aggregate(z, zeros_n, src3, dst3)
    return _mlp_c(q, b3, W4, b4)


# speedup vs baseline: 15.4142x; 1.0128x over previous
import functools

import jax
import jax.numpy as jnp
from jax import lax
from jax.experimental import pallas as pl
from jax.experimental.pallas import tpu as pltpu
from jax.experimental.pallas import tpu_sc as plsc

_N = 10000
_D = 128
_H = 64
_E = 320000

_NC = 2
_NS = 16
_NW = _NC * _NS
_CW = 128
_NCHT = _E // _CW
_CPW = 80
_NPADCH = _NW * _CPW - _NCHT
_NDUMP = 8
_NIT = 10
_RPT = _N // _NIT


def _sc_aggregate(y, zeros_n, src2, dst2):
    mesh = plsc.VectorSubcoreMesh(
        core_axis_name="c", subcore_axis_name="s", num_cores=_NC, num_subcores=_NS
    )
    nacc = _N + _NDUMP

    @functools.partial(
        pl.kernel,
        mesh=mesh,
        out_type=jax.ShapeDtypeStruct((_NC, _N, _H), jnp.float32),
        scratch_types=[
            pltpu.VMEM((_CPW, _CW), jnp.int32),
            pltpu.VMEM((_CPW, _CW), jnp.int32),
            pltpu.VMEM((2, _CW, _H), jnp.float32),
            pltpu.VMEM_SHARED((nacc, _H), jnp.float32),
            pltpu.SemaphoreType.DMA,
            pltpu.SemaphoreType.DMA,
            pltpu.SemaphoreType.DMA,
            pltpu.SemaphoreType.DMA,
        ],
        compiler_params=pltpu.CompilerParams(use_tc_tiling_on_sc=False),
    )
    def agg(y_hbm, zero_hbm, src_hbm, dst_hbm, out_hbm, src_v, dst_v, rows_v, acc, g0, g1, s0, s1):
        c = lax.axis_index("c")
        s = lax.axis_index("s")
        w = s * _NC + c
        r0 = s * _RPT
        c0 = w * _CPW

        pltpu.sync_copy(src_hbm.at[pl.ds(c0, _CPW)], src_v)
        pltpu.sync_copy(dst_hbm.at[pl.ds(c0, _CPW)], dst_v)

        @pl.when(jnp.logical_and(c == 0, s < _NIT))
        def _():
            pltpu.sync_copy(y_hbm.at[pl.ds(r0, _RPT)], acc.at[pl.ds(r0, _RPT)])

        @pl.when(jnp.logical_and(c == 1, s < _NIT))
        def _():
            pltpu.sync_copy(zero_hbm.at[pl.ds(r0, _RPT)], acc.at[pl.ds(r0, _RPT)])

        plsc.subcore_barrier()

        def g_start(j, slot, sem):
            return pltpu.async_copy(y_hbm.at[src_v.at[j]], rows_v.at[slot], sem)

        def g_wait(j, slot, sem):
            pltpu.make_async_copy(y_hbm.at[src_v.at[j]], rows_v.at[slot], sem).wait()

        def s_start(j, slot, sem):
            return pltpu.async_copy(rows_v.at[slot], acc.at[dst_v.at[j]], sem, add=True)

        def s_wait(j, slot, sem):
            pltpu.make_async_copy(rows_v.at[slot], acc.at[dst_v.at[j]], sem).wait()

        g_start(0, 0, g0)

        def body(g, carry):
            j0 = 2 * g
            j1 = j0 + 1

            @pl.when(g > 0)
            def _():
                s_wait(j1 - 2, 1, s1)

            g_start(j1, 1, g1)
            g_wait(j0, 0, g0)
            s_start(j0, 0, s0)
            s_wait(j0, 0, s0)

            @pl.when(g < _CPW // 2 - 1)
            def _():
                g_start(j0 + 2, 0, g0)

            g_wait(j1, 1, g1)
            s_start(j1, 1, s1)
            return carry

        lax.fori_loop(0, _CPW // 2, body, 0)
        s_wait(_CPW - 1, 1, s1)

        plsc.subcore_barrier()

        @pl.when(s < _NIT)
        def _():
            pltpu.sync_copy(acc.at[pl.ds(r0, _RPT)], out_hbm.at[c, pl.ds(r0, _RPT)])

    return agg(y, zeros_n, src2, dst2)


def _mm_a(x, w1):
    def body(x_ref, w_ref, o_ref):
        o_ref[...] = jnp.dot(x_ref[...], w_ref[...], preferred_element_type=jnp.float32)

    return pl.pallas_call(
        body,
        out_shape=jax.ShapeDtypeStruct((_N, _H), jnp.float32),
    )(x, w1)


def _mlp_b(p, b1, w2, b2, w3):
    def body(p_ref, b1_ref, w2_ref, b2_ref, w3_ref, o_ref):
        h = jnp.maximum(p_ref[0] + p_ref[1] + b1_ref[...], 0.0)
        h = jnp.maximum(
            jnp.dot(h, w2_ref[...], preferred_element_type=jnp.float32) + b2_ref[...], 0.0
        )
        o_ref[...] = jnp.dot(h, w3_ref[...], preferred_element_type=jnp.float32)

    return pl.pallas_call(
        body,
        out_shape=jax.ShapeDtypeStruct((_N, _H), jnp.float32),
    )(p, b1.reshape(1, _H), w2, b2.reshape(1, _H), w3)


def _mlp_c(q, b3, w4, b4):
    def body(q_ref, b3_ref, w4_ref, b4_ref, o_ref):
        h = jnp.maximum(q_ref[0] + q_ref[1] + b3_ref[...], 0.0)
        o_ref[...] = (
            jnp.dot(h, w4_ref[...], preferred_element_type=jnp.float32) + b4_ref[...]
        )

    return pl.pallas_call(
        body,
        out_shape=jax.ShapeDtypeStruct((_N, _D), jnp.float32),
    )(q, b3.reshape(1, _H), w4, b4.reshape(1, _D))


def kernel(x, edge_index, W1, b1, W2, b2, W3, b3, W4, b4):
    lane = jnp.arange(_CW, dtype=jnp.int32)
    pad_src = jnp.broadcast_to(lane * 64, (_NPADCH, _CW))
    pad_dst = jnp.broadcast_to(_N + (lane & 7), (_NPADCH, _CW))
    src2 = jnp.concatenate([edge_index[0].reshape(_NCHT, _CW), pad_src], axis=0)
    dst2 = jnp.concatenate([edge_index[1].reshape(_NCHT, _CW), pad_dst], axis=0)
    zeros_n = jnp.zeros((_N, _H), jnp.float32)

    y = _mm_a(x, W1)
    p = _sc_aggregate(y, zeros_n, src2, dst2)
    z = _mlp_b(p, b1, W2, b2, W3)
    q = _sc_aggregate(z, zeros_n, src2, dst2)
    return _mlp_c(q, b3, W4, b4)


# speedup vs baseline: 17.0685x; 1.1073x over previous
import functools

import jax
import jax.numpy as jnp
from jax import lax
from jax.experimental import pallas as pl
from jax.experimental.pallas import tpu as pltpu
from jax.experimental.pallas import tpu_sc as plsc

_N = 10000
_D = 128
_H = 64
_E = 320000

_NC = 2
_NS = 16
_NW = _NC * _NS
_CW = 512
_NCHT = _E // _CW
_CPW = 20
_NPADCH = _NW * _CPW - _NCHT
_NDUMP = 8
_NIT = 10
_RPT = _N // _NIT


def _sc_aggregate(y, zeros_n, src2, dst2):
    mesh = plsc.VectorSubcoreMesh(
        core_axis_name="c", subcore_axis_name="s", num_cores=_NC, num_subcores=_NS
    )
    nacc = _N + _NDUMP

    @functools.partial(
        pl.kernel,
        mesh=mesh,
        out_type=jax.ShapeDtypeStruct((_NC, _N, _H), jnp.float32),
        scratch_types=[
            pltpu.VMEM((_CPW, _CW), jnp.int32),
            pltpu.VMEM((_CPW, _CW), jnp.int32),
            pltpu.VMEM((2, _CW, _H), jnp.float32),
            pltpu.VMEM_SHARED((nacc, _H), jnp.float32),
            pltpu.SemaphoreType.DMA,
            pltpu.SemaphoreType.DMA,
            pltpu.SemaphoreType.DMA,
            pltpu.SemaphoreType.DMA,
        ],
        compiler_params=pltpu.CompilerParams(use_tc_tiling_on_sc=False),
    )
    def agg(y_hbm, zero_hbm, src_hbm, dst_hbm, out_hbm, src_v, dst_v, rows_v, acc, g0, g1, s0, s1):
        c = lax.axis_index("c")
        s = lax.axis_index("s")
        w = s * _NC + c
        r0 = s * _RPT

        pltpu.sync_copy(src_hbm.at[w], src_v)
        pltpu.sync_copy(dst_hbm.at[w], dst_v)

        @pl.when(jnp.logical_and(c == 0, s < _NIT))
        def _():
            pltpu.sync_copy(y_hbm.at[pl.ds(r0, _RPT)], acc.at[pl.ds(r0, _RPT)])

        @pl.when(jnp.logical_and(c == 1, s < _NIT))
        def _():
            pltpu.sync_copy(zero_hbm.at[pl.ds(r0, _RPT)], acc.at[pl.ds(r0, _RPT)])

        plsc.subcore_barrier()

        def g_start(j, slot, sem):
            return pltpu.async_copy(y_hbm.at[src_v.at[j]], rows_v.at[slot], sem)

        def g_wait(j, slot, sem):
            pltpu.make_async_copy(y_hbm.at[src_v.at[j]], rows_v.at[slot], sem).wait()

        def s_start(j, slot, sem):
            return pltpu.async_copy(rows_v.at[slot], acc.at[dst_v.at[j]], sem, add=True)

        def s_wait(j, slot, sem):
            pltpu.make_async_copy(rows_v.at[slot], acc.at[dst_v.at[j]], sem).wait()

        g_start(0, 0, g0)

        def body(g, carry):
            j0 = 2 * g
            j1 = j0 + 1

            @pl.when(g > 0)
            def _():
                s_wait(j1 - 2, 1, s1)

            g_start(j1, 1, g1)
            g_wait(j0, 0, g0)
            s_start(j0, 0, s0)
            s_wait(j0, 0, s0)

            @pl.when(g < _CPW // 2 - 1)
            def _():
                g_start(j0 + 2, 0, g0)

            g_wait(j1, 1, g1)
            s_start(j1, 1, s1)
            return carry

        lax.fori_loop(0, _CPW // 2, body, 0)
        s_wait(_CPW - 1, 1, s1)

        plsc.subcore_barrier()

        @pl.when(s < _NIT)
        def _():
            pltpu.sync_copy(acc.at[pl.ds(r0, _RPT)], out_hbm.at[c, pl.ds(r0, _RPT)])

    return agg(y, zeros_n, src2, dst2)


def _mm_a(x, w1):
    def body(x_ref, w_ref, o_ref):
        o_ref[...] = jnp.dot(x_ref[...], w_ref[...], preferred_element_type=jnp.float32)

    return pl.pallas_call(
        body,
        out_shape=jax.ShapeDtypeStruct((_N, _H), jnp.float32),
    )(x, w1)


def _mlp_b(p, b1, w2, b2, w3):
    def body(p_ref, b1_ref, w2_ref, b2_ref, w3_ref, o_ref):
        h = jnp.maximum(p_ref[0] + p_ref[1] + b1_ref[...], 0.0)
        h = jnp.maximum(
            jnp.dot(h, w2_ref[...], preferred_element_type=jnp.float32) + b2_ref[...], 0.0
        )
        o_ref[...] = jnp.dot(h, w3_ref[...], preferred_element_type=jnp.float32)

    return pl.pallas_call(
        body,
        out_shape=jax.ShapeDtypeStruct((_N, _H), jnp.float32),
    )(p, b1.reshape(1, _H), w2, b2.reshape(1, _H), w3)


def _mlp_c(q, b3, w4, b4):
    def body(q_ref, b3_ref, w4_ref, b4_ref, o_ref):
        h = jnp.maximum(q_ref[0] + q_ref[1] + b3_ref[...], 0.0)
        o_ref[...] = (
            jnp.dot(h, w4_ref[...], preferred_element_type=jnp.float32) + b4_ref[...]
        )

    return pl.pallas_call(
        body,
        out_shape=jax.ShapeDtypeStruct((_N, _D), jnp.float32),
    )(q, b3.reshape(1, _H), w4, b4.reshape(1, _D))


def kernel(x, edge_index, W1, b1, W2, b2, W3, b3, W4, b4):
    lane = jnp.arange(_CW, dtype=jnp.int32)
    pad_src = jnp.broadcast_to(lane * 16, (_NPADCH, _CW)).reshape(-1)
    pad_dst = jnp.broadcast_to(_N + (lane & 7), (_NPADCH, _CW)).reshape(-1)
    src2 = jnp.concatenate([edge_index[0], pad_src]).reshape(_NW, _CPW, _CW)
    dst2 = jnp.concatenate([edge_index[1], pad_dst]).reshape(_NW, _CPW, _CW)
    zeros_n = jnp.zeros((_N, _H), jnp.float32)

    y = _mm_a(x, W1)
    p = _sc_aggregate(y, zeros_n, src2, dst2)
    z = _mlp_b(p, b1, W2, b2, W3)
    q = _sc_aggregate(z, zeros_n, src2, dst2)
    return _mlp_c(q, b3, W4, b4)


# speedup vs baseline: 19.2117x; 1.1256x over previous
import functools

import jax
import jax.numpy as jnp
from jax import lax
from jax.experimental import pallas as pl
from jax.experimental.pallas import tpu as pltpu
from jax.experimental.pallas import tpu_sc as plsc

_N = 10000
_D = 128
_H = 64
_E = 320000

_NC = 2
_NS = 16
_NW = _NC * _NS
_CW = 512
_NCHT = _E // _CW
_CPW = 20
_NCHP = _NW * _CPW
_STG = 24
_NDUMP = 8
_NIT = 10
_RPT = _N // _NIT


def _sc_aggregate(y, zeros_n, src2, dst2):
    mesh = plsc.VectorSubcoreMesh(
        core_axis_name="c", subcore_axis_name="s", num_cores=_NC, num_subcores=_NS
    )
    nacc = _N + _NDUMP

    @functools.partial(
        pl.kernel,
        mesh=mesh,
        out_type=jax.ShapeDtypeStruct((_NC, _N, _H), jnp.float32),
        scratch_types=[
            pltpu.VMEM((_STG, _CW), jnp.int32),
            pltpu.VMEM((_STG, _CW), jnp.int32),
            pltpu.VMEM((2, _CW, _H), jnp.float32),
            pltpu.VMEM_SHARED((nacc, _H), jnp.float32),
            pltpu.SemaphoreType.DMA,
            pltpu.SemaphoreType.DMA,
            pltpu.SemaphoreType.DMA,
            pltpu.SemaphoreType.DMA,
        ],
        compiler_params=pltpu.CompilerParams(use_tc_tiling_on_sc=False),
    )
    def agg(y_hbm, zero_hbm, src_hbm, dst_hbm, out_hbm, src_v, dst_v, rows_v, acc, g0, g1, s0, s1):
        c = lax.axis_index("c")
        s = lax.axis_index("s")
        w = s * _NC + c
        r0 = s * _RPT

        base = w * _CPW
        a0 = base - lax.rem(base, 8)
        off = base - a0
        pltpu.sync_copy(src_hbm.at[pl.ds(a0, _STG)], src_v)
        pltpu.sync_copy(dst_hbm.at[pl.ds(a0, _STG)], dst_v)

        @pl.when(jnp.logical_and(c == 0, s < _NIT))
        def _():
            pltpu.sync_copy(y_hbm.at[pl.ds(r0, _RPT)], acc.at[pl.ds(r0, _RPT)])

        @pl.when(jnp.logical_and(c == 1, s < _NIT))
        def _():
            pltpu.sync_copy(zero_hbm.at[pl.ds(r0, _RPT)], acc.at[pl.ds(r0, _RPT)])

        plsc.subcore_barrier()

        def g_start(j, slot, sem):
            return pltpu.async_copy(y_hbm.at[src_v.at[off + j]], rows_v.at[slot], sem)

        def g_wait(j, slot, sem):
            pltpu.make_async_copy(y_hbm.at[src_v.at[off + j]], rows_v.at[slot], sem).wait()

        def s_start(j, slot, sem):
            return pltpu.async_copy(rows_v.at[slot], acc.at[dst_v.at[off + j]], sem, add=True)

        def s_wait(j, slot, sem):
            pltpu.make_async_copy(rows_v.at[slot], acc.at[dst_v.at[off + j]], sem).wait()

        g_start(0, 0, g0)

        def body(g, carry):
            j0 = 2 * g
            j1 = j0 + 1

            @pl.when(g > 0)
            def _():
                s_wait(j1 - 2, 1, s1)

            g_start(j1, 1, g1)
            g_wait(j0, 0, g0)
            s_start(j0, 0, s0)
            s_wait(j0, 0, s0)

            @pl.when(g < _CPW // 2 - 1)
            def _():
                g_start(j0 + 2, 0, g0)

            g_wait(j1, 1, g1)
            s_start(j1, 1, s1)
            return carry

        lax.fori_loop(0, _CPW // 2, body, 0)
        s_wait(_CPW - 1, 1, s1)

        plsc.subcore_barrier()

        @pl.when(s < _NIT)
        def _():
            pltpu.sync_copy(acc.at[pl.ds(r0, _RPT)], out_hbm.at[c, pl.ds(r0, _RPT)])

    return agg(y, zeros_n, src2, dst2)


def _bdiag(w):
    a, b = w.shape
    z = jnp.zeros((a, b), w.dtype)
    return jnp.concatenate(
        [jnp.concatenate([w, z], axis=1), jnp.concatenate([z, w], axis=1)], axis=0
    )


def _mm_a(x, w1):
    def body(x_ref, w_ref, o_ref):
        o_ref[...] = jnp.dot(x_ref[...], w_ref[...], preferred_element_type=jnp.float32)

    return pl.pallas_call(
        body,
        out_shape=jax.ShapeDtypeStruct((_N, _H), jnp.float32),
    )(x, w1)


def _mlp_b(p2, b1x, w2x, b2x, w3x):
    def body(p_ref, b1_ref, w2_ref, b2_ref, w3_ref, o_ref):
        h = jnp.maximum(p_ref[0] + p_ref[1] + b1_ref[...], 0.0)
        h = jnp.maximum(
            jnp.dot(h, w2_ref[...], preferred_element_type=jnp.float32) + b2_ref[...], 0.0
        )
        o_ref[...] = jnp.dot(h, w3_ref[...], preferred_element_type=jnp.float32)

    return pl.pallas_call(
        body,
        out_shape=jax.ShapeDtypeStruct((_N // 2, 2 * _H), jnp.float32),
    )(p2, b1x, w2x, b2x, w3x)


def _mlp_c(q2, b3x, w4x, b4x):
    def body(q_ref, b3_ref, w4_ref, b4_ref, o_ref):
        h = jnp.maximum(q_ref[0] + q_ref[1] + b3_ref[...], 0.0)
        o_ref[...] = (
            jnp.dot(h, w4_ref[...], preferred_element_type=jnp.float32) + b4_ref[...]
        )

    return pl.pallas_call(
        body,
        out_shape=jax.ShapeDtypeStruct((_N // 2, 2 * _D), jnp.float32),
    )(q2, b3x, w4x, b4x)


def kernel(x, edge_index, W1, b1, W2, b2, W3, b3, W4, b4):
    lane = jnp.arange(_CW, dtype=jnp.int32)
    npad = _NCHP - _NCHT
    pad_src = jnp.broadcast_to(lane * 16, (npad, _CW)).reshape(-1)
    pad_dst = jnp.broadcast_to(_N + (lane & 7), (npad, _CW)).reshape(-1)
    src2 = jnp.concatenate([edge_index[0], pad_src]).reshape(_NCHP, _CW)
    dst2 = jnp.concatenate([edge_index[1], pad_dst]).reshape(_NCHP, _CW)
    zeros_n = jnp.zeros((_N, _H), jnp.float32)

    b1x = jnp.concatenate([b1, b1]).reshape(1, 2 * _H)
    b2x = jnp.concatenate([b2, b2]).reshape(1, 2 * _H)
    b3x = jnp.concatenate([b3, b3]).reshape(1, 2 * _H)
    b4x = jnp.concatenate([b4, b4]).reshape(1, 2 * _D)
    w2x = _bdiag(W2)
    w3x = _bdiag(W3)
    w4x = _bdiag(W4)

    y = _mm_a(x, W1)
    p = _sc_aggregate(y, zeros_n, src2, dst2)
    z2 = _mlp_b(p.reshape(_NC, _N // 2, 2 * _H), b1x, w2x, b2x, w3x)
    q = _sc_aggregate(z2.reshape(_N, _H), zeros_n, src2, dst2)
    out2 = _mlp_c(q.reshape(_NC, _N // 2, 2 * _H), b3x, w4x, b4x)
    return out2.reshape(_N, _D)


# speedup vs baseline: 20.2189x; 1.0524x over previous
import functools

import jax
import jax.numpy as jnp
from jax import lax
from jax.experimental import pallas as pl
from jax.experimental.pallas import tpu as pltpu
from jax.experimental.pallas import tpu_sc as plsc

_N = 10000
_D = 128
_H = 64
_E = 320000

_NC = 2
_NS = 16
_NW = _NC * _NS
_CW = 512
_NCHT = _E // _CW
_CPW = 20
_NCHP = _NW * _CPW
_STG = 24
_NDUMP = 8
_NIT = 10
_RPT = _N // _NIT


def _sc_aggregate(y, zeros_n, src2, dst2):
    mesh = plsc.VectorSubcoreMesh(
        core_axis_name="c", subcore_axis_name="s", num_cores=_NC, num_subcores=_NS
    )
    nacc = _N + _NDUMP

    @functools.partial(
        pl.kernel,
        mesh=mesh,
        out_type=jax.ShapeDtypeStruct((_NC, _N, _H), jnp.float32),
        scratch_types=[
            pltpu.VMEM((_STG, _CW), jnp.int32),
            pltpu.VMEM((_STG, _CW), jnp.int32),
            pltpu.VMEM((2, _CW, _H), jnp.float32),
            pltpu.VMEM_SHARED((nacc, _H), jnp.float32),
            pltpu.SemaphoreType.DMA,
            pltpu.SemaphoreType.DMA,
            pltpu.SemaphoreType.DMA,
            pltpu.SemaphoreType.DMA,
        ],
        compiler_params=pltpu.CompilerParams(use_tc_tiling_on_sc=False),
    )
    def agg(y_hbm, zero_hbm, src_hbm, dst_hbm, out_hbm, src_v, dst_v, rows_v, acc, g0, g1, s0, s1):
        c = lax.axis_index("c")
        s = lax.axis_index("s")
        w = s * _NC + c
        r0 = s * _RPT

        base = w * _CPW
        a0 = base - lax.rem(base, 8)
        off = base - a0
        pltpu.sync_copy(src_hbm.at[pl.ds(a0, _STG)], src_v)
        pltpu.sync_copy(dst_hbm.at[pl.ds(a0, _STG)], dst_v)

        @pl.when(jnp.logical_and(c == 0, s < _NIT))
        def _():
            pltpu.sync_copy(y_hbm.at[pl.ds(r0, _RPT)], acc.at[pl.ds(r0, _RPT)])

        @pl.when(jnp.logical_and(c == 1, s < _NIT))
        def _():
            pltpu.sync_copy(zero_hbm.at[pl.ds(r0, _RPT)], acc.at[pl.ds(r0, _RPT)])

        plsc.subcore_barrier()

        def g_start(j, slot, sem):
            return pltpu.async_copy(y_hbm.at[src_v.at[off + j]], rows_v.at[slot], sem)

        def g_wait(j, slot, sem):
            pltpu.make_async_copy(y_hbm.at[src_v.at[off + j]], rows_v.at[slot], sem).wait()

        def s_start(j, slot, sem):
            return pltpu.async_copy(rows_v.at[slot], acc.at[dst_v.at[off + j]], sem, add=True)

        def s_wait(j, slot, sem):
            pltpu.make_async_copy(rows_v.at[slot], acc.at[dst_v.at[off + j]], sem).wait()

        g_start(0, 0, g0)

        def body(g, carry):
            j0 = 2 * g
            j1 = j0 + 1

            @pl.when(g > 0)
            def _():
                s_wait(j1 - 2, 1, s1)

            g_start(j1, 1, g1)
            g_wait(j0, 0, g0)
            s_start(j0, 0, s0)
            s_wait(j0, 0, s0)

            @pl.when(g < _CPW // 2 - 1)
            def _():
                g_start(j0 + 2, 0, g0)

            g_wait(j1, 1, g1)
            s_start(j1, 1, s1)
            return carry

        lax.fori_loop(0, _CPW // 2, body, 0)
        s_wait(_CPW - 1, 1, s1)

        plsc.subcore_barrier()

        @pl.when(s < _NIT)
        def _():
            pltpu.sync_copy(acc.at[pl.ds(r0, _RPT)], out_hbm.at[c, pl.ds(r0, _RPT)])

    return agg(y, zeros_n, src2, dst2)


def _bdiag(w):
    a, b = w.shape
    z = jnp.zeros((a, b), w.dtype)
    return jnp.concatenate(
        [jnp.concatenate([w, z], axis=1), jnp.concatenate([z, w], axis=1)], axis=0
    )


def _mm_a(x, w1):
    def body(x_ref, w_ref, o_ref):
        ya = jnp.dot(x_ref[: _N // 2], w_ref[...], preferred_element_type=jnp.float32)
        yb = jnp.dot(x_ref[_N // 2 :], w_ref[...], preferred_element_type=jnp.float32)
        o_ref[...] = jnp.concatenate([ya, yb], axis=1)

    return pl.pallas_call(
        body,
        out_shape=jax.ShapeDtypeStruct((_N // 2, 2 * _H), jnp.float32),
    )(x, w1)


def _mlp_b(p2, b1x, w2x, b2x, w3x):
    def body(p_ref, b1_ref, w2_ref, b2_ref, w3_ref, o_ref):
        h = jnp.maximum(p_ref[0] + p_ref[1] + b1_ref[...], 0.0)
        h = jnp.maximum(
            jnp.dot(h, w2_ref[...], preferred_element_type=jnp.float32) + b2_ref[...], 0.0
        )
        o_ref[...] = jnp.dot(h, w3_ref[...], preferred_element_type=jnp.float32)

    return pl.pallas_call(
        body,
        out_shape=jax.ShapeDtypeStruct((_N // 2, 2 * _H), jnp.float32),
    )(p2, b1x, w2x, b2x, w3x)


def _mlp_c(q2, b3x, w4, b4r):
    def body(q_ref, b3_ref, w4_ref, b4_ref, o_ref):
        h = jnp.maximum(q_ref[0] + q_ref[1] + b3_ref[...], 0.0)
        o_ref[: _N // 2] = (
            jnp.dot(h[:, : _H], w4_ref[...], preferred_element_type=jnp.float32)
            + b4_ref[...]
        )
        o_ref[_N // 2 :] = (
            jnp.dot(h[:, _H :], w4_ref[...], preferred_element_type=jnp.float32)
            + b4_ref[...]
        )

    return pl.pallas_call(
        body,
        out_shape=jax.ShapeDtypeStruct((_N, _D), jnp.float32),
    )(q2, b3x, w4, b4r)


def kernel(x, edge_index, W1, b1, W2, b2, W3, b3, W4, b4):
    lane = jnp.arange(_CW, dtype=jnp.int32)
    npad = _NCHP - _NCHT
    src = edge_index[0]
    dst = edge_index[1]
    phi_src = src * 2 - jnp.where(src >= _N // 2, _N - 1, 0)
    phi_dst = dst * 2 - jnp.where(dst >= _N // 2, _N - 1, 0)
    pad_src = jnp.broadcast_to(lane * 16, (npad, _CW)).reshape(-1)
    pad_dst = jnp.broadcast_to(_N + (lane & 7), (npad, _CW)).reshape(-1)
    src2 = jnp.concatenate([phi_src, pad_src]).reshape(_NCHP, _CW)
    dst2 = jnp.concatenate([phi_dst, pad_dst]).reshape(_NCHP, _CW)
    zeros_n = jnp.zeros((_N, _H), jnp.float32)

    b1x = jnp.concatenate([b1, b1]).reshape(1, 2 * _H)
    b2x = jnp.concatenate([b2, b2]).reshape(1, 2 * _H)
    b3x = jnp.concatenate([b3, b3]).reshape(1, 2 * _H)
    w2x = _bdiag(W2)
    w3x = _bdiag(W3)

    y2 = _mm_a(x, W1)
    p = _sc_aggregate(y2.reshape(_N, _H), zeros_n, src2, dst2)
    z2 = _mlp_b(p.reshape(_NC, _N // 2, 2 * _H), b1x, w2x, b2x, w3x)
    q = _sc_aggregate(z2.reshape(_N, _H), zeros_n, src2, dst2)
    return _mlp_c(q.reshape(_NC, _N // 2, 2 * _H), b3x, W4, b4.reshape(1, _D))
